# baseline (speedup 1.0000x reference)
import functools

import jax
import jax.numpy as jnp
from jax import lax
from jax.experimental import pallas as pl
from jax.experimental.pallas import tpu as pltpu
from jax.experimental.pallas import tpu_sc as plsc

VOCAB = 1000000
EMBED = 64
C = 5
BATCH = 16384
NC = 2
NS = 16
NW = NC * NS
BPW = BATCH // NW
CB = 128
NCH = BPW // CB
L = 16
TBLK = 1024
PAIR = 2 * EMBED


def _detile_body(src_ref, dst_ref):
    eye = jnp.float32(
        lax.broadcasted_iota(jnp.int32, (EMBED, EMBED), 0)
        == lax.broadcasted_iota(jnp.int32, (EMBED, EMBED), 1))
    dst_ref[...] = lax.dot_general(
        src_ref[...], eye, (((0,), (0,)), ((), ())),
        preferred_element_type=jnp.float32)


_detile_tc = pl.pallas_call(
    _detile_body,
    grid=((VOCAB + TBLK - 1) // TBLK,),
    in_specs=[pl.BlockSpec((EMBED, TBLK), lambda i: (0, i))],
    out_specs=pl.BlockSpec((TBLK, EMBED), lambda i: (i, 0)),
    out_shape=jax.ShapeDtypeStruct((VOCAB, EMBED), jnp.float32),
)


def _make_sc_kernel():
    mesh = plsc.VectorSubcoreMesh(core_axis_name="c", subcore_axis_name="s")

    @functools.partial(
        pl.kernel,
        mesh=mesh,
        compiler_params=pltpu.CompilerParams(needs_layout_passes=False),
        out_type=jax.ShapeDtypeStruct((C, BATCH), jnp.float32),
        scratch_types=[
            pltpu.VMEM((CB,), jnp.int32),
            pltpu.VMEM((C, CB), jnp.int32),
            pltpu.VMEM((CB,), jnp.int32),
            pltpu.VMEM((C, CB), jnp.int32),
            pltpu.VMEM((CB, PAIR), jnp.float32),
            pltpu.VMEM((C, CB, PAIR), jnp.float32),
            pltpu.VMEM((C, CB), jnp.float32),
            pltpu.VMEM((C * L * L,), jnp.float32),
            pltpu.SemaphoreType.DMA,
        ],
    )
    def word2vec_sc(tgt_hbm, ctx_hbm, ttab_hbm, ctab_hbm, out_hbm,
                    tidx, cidx, trid, crid, trows, crows, outv, pmat, sem):
        wid = lax.axis_index("s") * NC + lax.axis_index("c")

        def chunk_body(g, carry):
            base = wid * BPW + g * CB
            pltpu.sync_copy(tgt_hbm.at[pl.ds(base, CB)], tidx)
            pltpu.sync_copy(ctx_hbm.at[:, pl.ds(base, CB)], cidx)

            def rid_body(i16, carry2):
                col = i16 * L
                trid[pl.ds(col, L)] = tidx[pl.ds(col, L)] >> 1
                for c in range(C):
                    crid[c, pl.ds(col, L)] = cidx[c, pl.ds(col, L)] >> 1
                return carry2

            lax.fori_loop(0, CB // L, rid_body, 0)

            copies = [pltpu.async_copy(ttab_hbm.at[trid], trows, sem)]
            for j in range(C):
                copies.append(
                    pltpu.async_copy(ctab_hbm.at[crid.at[j]], crows.at[j], sem))
            for cp in copies:
                cp.wait()

            lane = lax.iota(jnp.int32, L)

            def group_body(i16, carry2):
                col = i16 * L
                tvec = tidx[pl.ds(col, L)]
                cvecs = [cidx[c, pl.ds(col, L)] for c in range(C)]
                for ii in range(L):
                    i = col + ii
                    toffs = (tvec[ii] & 1) * EMBED
                    we = [trows[i, pl.ds(toffs + k * L, L)]
                          for k in range(EMBED // L)]
                    for c in range(C):
                        coffs = (cvecs[c][ii] & 1) * EMBED
                        p = we[0] * crows[c, i, pl.ds(coffs, L)]
                        for k in range(1, EMBED // L):
                            p = p + we[k] * crows[c, i, pl.ds(coffs + k * L, L)]
                        plsc.store_scatter(pmat, [c * L * L + lane * L + ii], p)
                for c in range(C):
                    acc = pmat[pl.ds(c * L * L, L)]
                    for j in range(1, L):
                        acc = acc + pmat[pl.ds(c * L * L + j * L, L)]
                    outv[c, pl.ds(col, L)] = acc
                return carry2

            lax.fori_loop(0, CB // L, group_body, 0)
            pltpu.sync_copy(outv, out_hbm.at[:, pl.ds(base, CB)])
            return carry

        lax.fori_loop(0, NCH, chunk_body, 0)

    return word2vec_sc


_word2vec_sc = _make_sc_kernel()


@jax.jit
def kernel(target, context, target_table, context_table):
    tgt_flat = target.reshape(BATCH)
    ctx_t = context.T
    ttab_pair = _detile_tc(target_table.T).reshape(VOCAB // 2, PAIR)
    ctab_pair = _detile_tc(context_table.T).reshape(VOCAB // 2, PAIR)
    out_t = _word2vec_sc(tgt_flat, ctx_t, ttab_pair, ctab_pair)
    return out_t.T


# speedup vs baseline: 1.7011x; 1.7011x over previous
import functools

import jax
import jax.numpy as jnp
from jax import lax
from jax.experimental import pallas as pl
from jax.experimental.pallas import tpu as pltpu
from jax.experimental.pallas import tpu_sc as plsc

VOCAB = 1000000
EMBED = 64
C = 5
BATCH = 16384
NC = 2
NS = 16
NW = NC * NS
BPW = BATCH // NW
CB = 128
NCH = BPW // CB
L = 16
TBLK = 8192
PAIR = 2 * EMBED


def _detile_body(src_ref, dst_ref):
    eye = jnp.float32(
        lax.broadcasted_iota(jnp.int32, (EMBED, EMBED), 0)
        == lax.broadcasted_iota(jnp.int32, (EMBED, EMBED), 1))
    dst_ref[...] = lax.dot_general(
        src_ref[...], eye, (((0,), (0,)), ((), ())),
        preferred_element_type=jnp.float32)


_detile_tc = pl.pallas_call(
    _detile_body,
    grid=((VOCAB + TBLK - 1) // TBLK,),
    in_specs=[pl.BlockSpec((EMBED, TBLK), lambda i: (0, i))],
    out_specs=pl.BlockSpec((TBLK, EMBED), lambda i: (i, 0)),
    out_shape=jax.ShapeDtypeStruct((VOCAB, EMBED), jnp.float32),
)


def _make_sc_kernel():
    mesh = plsc.VectorSubcoreMesh(core_axis_name="c", subcore_axis_name="s")

    @functools.partial(
        pl.kernel,
        mesh=mesh,
        compiler_params=pltpu.CompilerParams(needs_layout_passes=False),
        out_type=jax.ShapeDtypeStruct((C, BATCH), jnp.float32),
        scratch_types=[
            pltpu.VMEM((CB,), jnp.int32),
            pltpu.VMEM((C, CB), jnp.int32),
            pltpu.VMEM((CB,), jnp.int32),
            pltpu.VMEM((C, CB), jnp.int32),
            pltpu.VMEM((CB, PAIR), jnp.float32),
            pltpu.VMEM((C, CB, PAIR), jnp.float32),
            pltpu.VMEM((C, CB), jnp.float32),
            pltpu.VMEM((C * L * L,), jnp.float32),
            pltpu.SemaphoreType.DMA,
        ],
    )
    def word2vec_sc(tgt_hbm, ctx_hbm, ttab_hbm, ctab_hbm, out_hbm,
                    tidx, cidx, trid, crid, trows, crows, outv, pmat, sem):
        wid = lax.axis_index("s") * NC + lax.axis_index("c")

        def chunk_body(g, carry):
            base = wid * BPW + g * CB
            pltpu.sync_copy(tgt_hbm.at[pl.ds(base, CB)], tidx)
            pltpu.sync_copy(ctx_hbm.at[:, pl.ds(base, CB)], cidx)

            def rid_body(i16, carry2):
                col = i16 * L
                trid[pl.ds(col, L)] = tidx[pl.ds(col, L)] >> 1
                for c in range(C):
                    crid[c, pl.ds(col, L)] = cidx[c, pl.ds(col, L)] >> 1
                return carry2

            lax.fori_loop(0, CB // L, rid_body, 0)

            copies = [pltpu.async_copy(ttab_hbm.at[trid], trows, sem)]
            for j in range(C):
                copies.append(
                    pltpu.async_copy(ctab_hbm.at[crid.at[j]], crows.at[j], sem))
            for cp in copies:
                cp.wait()

            lane = lax.iota(jnp.int32, L)

            def group_body(i16, carry2):
                col = i16 * L
                tvec = tidx[pl.ds(col, L)]
                cvecs = [cidx[c, pl.ds(col, L)] for c in range(C)]
                for ii in range(L):
                    i = col + ii
                    toffs = (tvec[ii] & 1) * EMBED
                    we = [trows[i, pl.ds(toffs + k * L, L)]
                          for k in range(EMBED // L)]
                    for c in range(C):
                        coffs = (cvecs[c][ii] & 1) * EMBED
                        p = we[0] * crows[c, i, pl.ds(coffs, L)]
                        for k in range(1, EMBED // L):
                            p = p + we[k] * crows[c, i, pl.ds(coffs + k * L, L)]
                        plsc.store_scatter(pmat, [c * L * L + lane * L + ii], p)
                for c in range(C):
                    acc = pmat[pl.ds(c * L * L, L)]
                    for j in range(1, L):
                        acc = acc + pmat[pl.ds(c * L * L + j * L, L)]
                    outv[c, pl.ds(col, L)] = acc
                return carry2

            lax.fori_loop(0, CB // L, group_body, 0)
            pltpu.sync_copy(outv, out_hbm.at[:, pl.ds(base, CB)])
            return carry

        lax.fori_loop(0, NCH, chunk_body, 0)

    return word2vec_sc


_word2vec_sc = _make_sc_kernel()


@jax.jit
def kernel(target, context, target_table, context_table):
    tgt_flat = target.reshape(BATCH)
    ctx_t = context.T
    ttab_pair = _detile_tc(target_table.T).reshape(VOCAB // 2, PAIR)
    ctab_pair = _detile_tc(context_table.T).reshape(VOCAB // 2, PAIR)
    out_t = _word2vec_sc(tgt_flat, ctx_t, ttab_pair, ctab_pair)
    return out_t.T


# speedup vs baseline: 2.0203x; 1.1877x over previous
import functools

import jax
import jax.numpy as jnp
from jax import lax
from jax.experimental import pallas as pl
from jax.experimental.pallas import tpu as pltpu
from jax.experimental.pallas import tpu_sc as plsc

VOCAB = 1000000
EMBED = 64
C = 5
BATCH = 16384
NC = 2
NS = 16
NW = NC * NS
BPW = BATCH // NW
CB = 128
NCH = BPW // CB
L = 16


def _make_kernel():
    mesh = plsc.VectorSubcoreMesh(core_axis_name="c", subcore_axis_name="s")

    @functools.partial(
        pl.kernel,
        mesh=mesh,
        compiler_params=pltpu.CompilerParams(
            needs_layout_passes=False, use_tc_tiling_on_sc=False),
        out_type=jax.ShapeDtypeStruct((BATCH * C,), jnp.float32),
        scratch_types=[
            pltpu.VMEM((CB,), jnp.int32),
            pltpu.VMEM((C, CB), jnp.int32),
            pltpu.VMEM((CB, EMBED), jnp.float32),
            pltpu.VMEM((C, CB, EMBED), jnp.float32),
            pltpu.VMEM((CB * C,), jnp.float32),
            pltpu.VMEM((C * L * L,), jnp.float32),
            pltpu.SemaphoreType.DMA,
        ],
    )
    def word2vec_sc(tgt_hbm, ctx_hbm, ttab_hbm, ctab_hbm, out_hbm,
                    tidx, cidx, trows, crows, outv, pmat, sem):
        wid = lax.axis_index("s") * NC + lax.axis_index("c")

        def chunk_body(g, carry):
            base = wid * BPW + g * CB
            pltpu.sync_copy(tgt_hbm.at[pl.ds(base, CB)], tidx)
            pltpu.sync_copy(ctx_hbm.at[:, pl.ds(base, CB)], cidx)
            copies = [pltpu.async_copy(ttab_hbm.at[tidx], trows, sem)]
            for j in range(C):
                copies.append(
                    pltpu.async_copy(ctab_hbm.at[cidx.at[j]], crows.at[j], sem))
            for cp in copies:
                cp.wait()

            lane = lax.iota(jnp.int32, L)

            def group_body(i16, carry2):
                for ii in range(L):
                    i = i16 * L + ii
                    we = [trows[i, pl.ds(k * L, L)] for k in range(EMBED // L)]
                    for c in range(C):
                        p = we[0] * crows[c, i, pl.ds(0, L)]
                        for k in range(1, EMBED // L):
                            p = p + we[k] * crows[c, i, pl.ds(k * L, L)]
                        plsc.store_scatter(pmat, [c * L * L + lane * L + ii], p)
                for c in range(C):
                    acc = pmat[pl.ds(c * L * L, L)]
                    for j in range(1, L):
                        acc = acc + pmat[pl.ds(c * L * L + j * L, L)]
                    plsc.store_scatter(
                        outv, [(i16 * L + lane) * C + c], acc)
                return carry2

            lax.fori_loop(0, CB // L, group_body, 0)
            pltpu.sync_copy(outv, out_hbm.at[pl.ds(base * C, CB * C)])
            return carry

        lax.fori_loop(0, NCH, chunk_body, 0)

    return word2vec_sc


_word2vec_sc = _make_kernel()


@jax.jit
def kernel(target, context, target_table, context_table):
    tgt_flat = target.reshape(BATCH)
    ctx_t = context.T
    out_flat = _word2vec_sc(tgt_flat, ctx_t, target_table, context_table)
    return out_flat.reshape(BATCH, C)
